# speedup vs baseline: 23.3453x; 23.3453x over previous
import jax
import jax.numpy as jnp
from jax.experimental import pallas as pl
from jax.experimental.pallas import tpu as pltpu

_B = 8
_N = 16384
_NPT = 512


def _transpose_col(col, dtype):
    r = jax.lax.broadcasted_iota(jnp.int32, (_B, _B), 0)
    c = jax.lax.broadcasted_iota(jnp.int32, (_B, _B), 1)
    ey = r == c
    mat = jnp.where(ey, jnp.broadcast_to(col, (_B, _B)), jnp.zeros((), dtype))
    return jnp.sum(mat, axis=0, keepdims=True)


def _fps_body(x_ref, y_ref, z_ref, far0_ref, idx_ref, cx_ref, cy_ref, cz_ref,
              dist_ref):
    x = x_ref[...]
    y = y_ref[...]
    z = z_ref[...]
    iota = jax.lax.broadcasted_iota(jnp.int32, (_B, _N), 1)
    dist_ref[...] = jnp.full((_B, _N), 1e10, jnp.float32)
    neg = jnp.float32(-jnp.inf)

    def body(i, far):
        sel = iota == far
        cx = jnp.max(jnp.where(sel, x, neg), axis=1, keepdims=True)
        cy = jnp.max(jnp.where(sel, y, neg), axis=1, keepdims=True)
        cz = jnp.max(jnp.where(sel, z, neg), axis=1, keepdims=True)
        dx = x - cx
        dy = y - cy
        dz = z - cz
        dnew = dx * dx + dy * dy + dz * dz
        d = dist_ref[...]
        d = jnp.where(dnew < d, dnew, d)
        dist_ref[...] = d
        m = jnp.max(d, axis=1, keepdims=True)
        nxt = jnp.min(jnp.where(d == m, iota, jnp.int32(_N)), axis=1,
                      keepdims=True)
        idx_ref[pl.ds(i, 1), :] = _transpose_col(far, jnp.int32)
        cx_ref[pl.ds(i, 1), :] = _transpose_col(cx, jnp.float32)
        cy_ref[pl.ds(i, 1), :] = _transpose_col(cy, jnp.float32)
        cz_ref[pl.ds(i, 1), :] = _transpose_col(cz, jnp.float32)
        return nxt

    jax.lax.fori_loop(0, _NPT, body, far0_ref[...])


@jax.jit
def kernel(xyz):
    x = xyz[:, :, 0]
    y = xyz[:, :, 1]
    z = xyz[:, :, 2]
    far0 = jax.random.randint(jax.random.key(1), (_B,), 0, _N,
                              dtype=jnp.int32).reshape(_B, 1)
    out_shapes = (
        jax.ShapeDtypeStruct((_NPT, _B), jnp.int32),
        jax.ShapeDtypeStruct((_NPT, _B), jnp.float32),
        jax.ShapeDtypeStruct((_NPT, _B), jnp.float32),
        jax.ShapeDtypeStruct((_NPT, _B), jnp.float32),
    )
    idx_t, cx_t, cy_t, cz_t = pl.pallas_call(
        _fps_body,
        out_shape=out_shapes,
        scratch_shapes=[pltpu.VMEM((_B, _N), jnp.float32)],
    )(x, y, z, far0)
    idx = idx_t.T
    new_xyz = jnp.stack([cx_t.T, cy_t.T, cz_t.T], axis=-1)
    return (new_xyz, idx)


# speedup vs baseline: 33.9998x; 1.4564x over previous
import jax
import jax.numpy as jnp
from jax.experimental import pallas as pl
from jax.experimental.pallas import tpu as pltpu

_B = 8
_N = 16384
_NPT = 512
_W = 128
_NSL = _N // _W
_NACC = 8


def _transpose_col(col, dtype):
    r = jax.lax.broadcasted_iota(jnp.int32, (_B, _B), 0)
    c = jax.lax.broadcasted_iota(jnp.int32, (_B, _B), 1)
    ey = r == c
    mat = jnp.where(ey, jnp.broadcast_to(col, (_B, _B)), jnp.zeros((), dtype))
    return jnp.sum(mat, axis=0, keepdims=True)


def _fps_body(x_ref, y_ref, z_ref, iota_ref, init_ref, idx_ref, cx_ref,
              cy_ref, cz_ref, dist_ref):
    dist_ref[...] = jnp.full((_B, _N), 1e10, jnp.float32)
    neg = jnp.float32(-jnp.inf)

    def body(i, carry):
        far, cx, cy, cz = carry
        idx_ref[pl.ds(i, 1), :] = _transpose_col(far, jnp.int32)
        cx_ref[pl.ds(i, 1), :] = _transpose_col(cx, jnp.float32)
        cy_ref[pl.ds(i, 1), :] = _transpose_col(cy, jnp.float32)
        cz_ref[pl.ds(i, 1), :] = _transpose_col(cz, jnp.float32)

        accs = [None] * _NACC
        for v in range(_NSL):
            sl = slice(v * _W, (v + 1) * _W)
            xv = x_ref[:, sl]
            yv = y_ref[:, sl]
            zv = z_ref[:, sl]
            iv = iota_ref[:, sl]
            dxv = xv - cx
            dyv = yv - cy
            dzv = zv - cz
            dn = dxv * dxv + dyv * dyv + dzv * dzv
            dv = jnp.minimum(dn, dist_ref[:, sl])
            dist_ref[:, sl] = dv
            k = v % _NACC
            if accs[k] is None:
                accs[k] = (dv, iv, xv, yv, zv)
            else:
                pv, pi, px, py, pz = accs[k]
                g = dv > pv
                accs[k] = (jnp.where(g, dv, pv), jnp.where(g, iv, pi),
                           jnp.where(g, xv, px), jnp.where(g, yv, py),
                           jnp.where(g, zv, pz))

        vals = jnp.concatenate([a[0] for a in accs], axis=1)
        idxs = jnp.concatenate([a[1] for a in accs], axis=1)
        xs = jnp.concatenate([a[2] for a in accs], axis=1)
        ys = jnp.concatenate([a[3] for a in accs], axis=1)
        zs = jnp.concatenate([a[4] for a in accs], axis=1)
        m = jnp.max(vals, axis=1, keepdims=True)
        eqm = vals == m
        nidx = jnp.min(jnp.where(eqm, idxs, jnp.int32(_N)), axis=1,
                       keepdims=True)
        sel2 = idxs == nidx
        ncx = jnp.max(jnp.where(sel2, xs, neg), axis=1, keepdims=True)
        ncy = jnp.max(jnp.where(sel2, ys, neg), axis=1, keepdims=True)
        ncz = jnp.max(jnp.where(sel2, zs, neg), axis=1, keepdims=True)
        return (nidx, ncx, ncy, ncz)

    far0 = jax.lax.bitcast_convert_type(init_ref[:, 0:1], jnp.int32)
    cx0 = init_ref[:, 1:2]
    cy0 = init_ref[:, 2:3]
    cz0 = init_ref[:, 3:4]
    jax.lax.fori_loop(0, _NPT, body, (far0, cx0, cy0, cz0))


@jax.jit
def kernel(xyz):
    x = xyz[:, :, 0]
    y = xyz[:, :, 1]
    z = xyz[:, :, 2]
    far0 = jax.random.randint(jax.random.key(1), (_B,), 0, _N,
                              dtype=jnp.int32)
    c0 = jnp.take_along_axis(xyz, far0[:, None, None], axis=1)[:, 0, :]
    init = jnp.concatenate(
        [jax.lax.bitcast_convert_type(far0, jnp.float32)[:, None], c0],
        axis=1)
    iota = jnp.broadcast_to(jnp.arange(_N, dtype=jnp.int32)[None, :],
                            (_B, _N))
    out_shapes = (
        jax.ShapeDtypeStruct((_NPT, _B), jnp.int32),
        jax.ShapeDtypeStruct((_NPT, _B), jnp.float32),
        jax.ShapeDtypeStruct((_NPT, _B), jnp.float32),
        jax.ShapeDtypeStruct((_NPT, _B), jnp.float32),
    )
    idx_t, cx_t, cy_t, cz_t = pl.pallas_call(
        _fps_body,
        out_shape=out_shapes,
        scratch_shapes=[pltpu.VMEM((_B, _N), jnp.float32)],
    )(x, y, z, iota, init)
    idx = idx_t.T
    new_xyz = jnp.stack([cx_t.T, cy_t.T, cz_t.T], axis=-1)
    return (new_xyz, idx)


# speedup vs baseline: 34.4930x; 1.0145x over previous
import jax
import jax.numpy as jnp
from jax.experimental import pallas as pl
from jax.experimental.pallas import tpu as pltpu

_B = 8
_N = 16384
_NPT = 512
_W = 128
_NSL = _N // _W
_NACC = 8


def _transpose_col(col, dtype):
    r = jax.lax.broadcasted_iota(jnp.int32, (_B, _B), 0)
    c = jax.lax.broadcasted_iota(jnp.int32, (_B, _B), 1)
    ey = r == c
    mat = jnp.where(ey, jnp.broadcast_to(col, (_B, _B)), jnp.zeros((), dtype))
    return jnp.sum(mat, axis=0, keepdims=True)


def _fps_body(x_ref, y_ref, z_ref, iota_ref, far0_ref, c0_ref, idx_ref,
              cx_ref, cy_ref, cz_ref, dist_ref):
    dist_ref[...] = jnp.full((_B, _N), 1e10, jnp.float32)
    neg = jnp.float32(-jnp.inf)

    def body(i, carry):
        far, cx, cy, cz = carry
        idx_ref[pl.ds(i, 1), :] = _transpose_col(far, jnp.int32)
        cx_ref[pl.ds(i, 1), :] = _transpose_col(cx, jnp.float32)
        cy_ref[pl.ds(i, 1), :] = _transpose_col(cy, jnp.float32)
        cz_ref[pl.ds(i, 1), :] = _transpose_col(cz, jnp.float32)

        accs = [None] * _NACC
        for v in range(_NSL):
            sl = slice(v * _W, (v + 1) * _W)
            xv = x_ref[:, sl]
            yv = y_ref[:, sl]
            zv = z_ref[:, sl]
            iv = iota_ref[:, sl]
            dxv = xv - cx
            dyv = yv - cy
            dzv = zv - cz
            dn = dxv * dxv + dyv * dyv + dzv * dzv
            dv = jnp.minimum(dn, dist_ref[:, sl])
            dist_ref[:, sl] = dv
            k = v % _NACC
            if accs[k] is None:
                accs[k] = (dv, iv, xv, yv, zv)
            else:
                pv, pi, px, py, pz = accs[k]
                g = dv > pv
                accs[k] = (jnp.where(g, dv, pv), jnp.where(g, iv, pi),
                           jnp.where(g, xv, px), jnp.where(g, yv, py),
                           jnp.where(g, zv, pz))

        vals = jnp.concatenate([a[0] for a in accs], axis=1)
        idxs = jnp.concatenate([a[1] for a in accs], axis=1)
        xs = jnp.concatenate([a[2] for a in accs], axis=1)
        ys = jnp.concatenate([a[3] for a in accs], axis=1)
        zs = jnp.concatenate([a[4] for a in accs], axis=1)
        m = jnp.max(vals, axis=1, keepdims=True)
        eqm = vals == m
        nidx = jnp.min(jnp.where(eqm, idxs, jnp.int32(_N)), axis=1,
                       keepdims=True)
        sel2 = idxs == nidx
        ncx = jnp.max(jnp.where(sel2, xs, neg), axis=1, keepdims=True)
        ncy = jnp.max(jnp.where(sel2, ys, neg), axis=1, keepdims=True)
        ncz = jnp.max(jnp.where(sel2, zs, neg), axis=1, keepdims=True)
        return (nidx, ncx, ncy, ncz)

    far0 = far0_ref[...]
    cx0 = c0_ref[:, 0:1]
    cy0 = c0_ref[:, 1:2]
    cz0 = c0_ref[:, 2:3]
    jax.lax.fori_loop(0, _NPT, body, (far0, cx0, cy0, cz0))


@jax.jit
def kernel(xyz):
    x = xyz[:, :, 0]
    y = xyz[:, :, 1]
    z = xyz[:, :, 2]
    far0 = jax.random.randint(jax.random.key(1), (_B,), 0, _N,
                              dtype=jnp.int32)
    c0 = jnp.take_along_axis(xyz, far0[:, None, None], axis=1)[:, 0, :]
    iota = jnp.broadcast_to(jnp.arange(_N, dtype=jnp.int32)[None, :],
                            (_B, _N))
    out_shapes = (
        jax.ShapeDtypeStruct((_NPT, _B), jnp.int32),
        jax.ShapeDtypeStruct((_NPT, _B), jnp.float32),
        jax.ShapeDtypeStruct((_NPT, _B), jnp.float32),
        jax.ShapeDtypeStruct((_NPT, _B), jnp.float32),
    )
    idx_t, cx_t, cy_t, cz_t = pl.pallas_call(
        _fps_body,
        out_shape=out_shapes,
        scratch_shapes=[pltpu.VMEM((_B, _N), jnp.float32)],
    )(x, y, z, iota, far0[:, None], c0)
    idx = idx_t.T
    new_xyz = jnp.stack([cx_t.T, cy_t.T, cz_t.T], axis=-1)
    return (new_xyz, idx)


# speedup vs baseline: 41.1830x; 1.1940x over previous
import jax
import jax.numpy as jnp
from jax.experimental import pallas as pl
from jax.experimental.pallas import tpu as pltpu

_B = 8
_N = 16384
_NPT = 512
_W = 128
_NSL = _N // _W
_NACC = 2


def _transpose_col(colf):
    r = jax.lax.broadcasted_iota(jnp.int32, (_B, _B), 0)
    c = jax.lax.broadcasted_iota(jnp.int32, (_B, _B), 1)
    ey = r == c
    mat = jnp.where(ey, colf[:, 0:_B], jnp.float32(0))
    return jnp.sum(mat, axis=0, keepdims=True)


def _fps_body(x_ref, y_ref, z_ref, iota_ref, tri_ref, far0_ref, c0_ref,
              idx_ref, cx_ref, cy_ref, cz_ref, dist_ref):
    dist_ref[...] = jnp.full((_B, _N), 1e10, jnp.float32)

    def body(i, carry):
        far, cx, cy, cz = carry
        idx_ref[pl.ds(i, 1), :] = _transpose_col(far)
        cx_ref[pl.ds(i, 1), :] = _transpose_col(cx)
        cy_ref[pl.ds(i, 1), :] = _transpose_col(cy)
        cz_ref[pl.ds(i, 1), :] = _transpose_col(cz)

        accs = [None] * _NACC
        for v in range(_NSL):
            sl = slice(v * _W, (v + 1) * _W)
            xv = x_ref[:, sl]
            yv = y_ref[:, sl]
            zv = z_ref[:, sl]
            iv = iota_ref[:, sl]
            dxv = xv - cx
            dyv = yv - cy
            dzv = zv - cz
            dn = dxv * dxv + dyv * dyv + dzv * dzv
            dv = jnp.minimum(dn, dist_ref[:, sl])
            dist_ref[:, sl] = dv
            k = v % _NACC
            if accs[k] is None:
                accs[k] = (dv, iv, xv, yv, zv)
            else:
                pv, pi, px, py, pz = accs[k]
                g = dv > pv
                accs[k] = (jnp.where(g, dv, pv), jnp.where(g, iv, pi),
                           jnp.where(g, xv, px), jnp.where(g, yv, py),
                           jnp.where(g, zv, pz))

        def combine(a, b):
            takeb = (b[0] > a[0]) | ((b[0] == a[0]) & (b[1] < a[1]))
            return tuple(jnp.where(takeb, bb, aa) for aa, bb in zip(a, b))

        t = accs[0]
        for a in accs[1:]:
            t = combine(t, a)
        vals, idxs, xs, ys, zs = t
        m = jnp.max(vals, axis=1, keepdims=True)
        eqm = vals == m
        cum = jnp.dot(eqm.astype(jnp.float32), tri_ref[...],
                      preferred_element_type=jnp.float32)
        first = eqm & (cum == 1.0)
        neg = jnp.float32(-jnp.inf)
        nfar = jnp.max(jnp.where(first, idxs, jnp.float32(-1.0)), axis=1,
                       keepdims=True)
        ncx = jnp.max(jnp.where(first, xs, neg), axis=1, keepdims=True)
        ncy = jnp.max(jnp.where(first, ys, neg), axis=1, keepdims=True)
        ncz = jnp.max(jnp.where(first, zs, neg), axis=1, keepdims=True)
        bto = lambda a: jnp.broadcast_to(a, (_B, _W))
        return (bto(nfar), bto(ncx), bto(ncy), bto(ncz))

    far0 = jnp.broadcast_to(far0_ref[...], (_B, _W))
    cx0 = jnp.broadcast_to(c0_ref[:, 0:1], (_B, _W))
    cy0 = jnp.broadcast_to(c0_ref[:, 1:2], (_B, _W))
    cz0 = jnp.broadcast_to(c0_ref[:, 2:3], (_B, _W))
    jax.lax.fori_loop(0, _NPT, body, (far0, cx0, cy0, cz0))


def _perm(a):
    return a.reshape(_B, _NSL, _W).transpose(0, 2, 1).reshape(_B, _N)


@jax.jit
def kernel(xyz):
    x = _perm(xyz[:, :, 0])
    y = _perm(xyz[:, :, 1])
    z = _perm(xyz[:, :, 2])
    far0 = jax.random.randint(jax.random.key(1), (_B,), 0, _N,
                              dtype=jnp.int32)
    c0 = jnp.take_along_axis(xyz, far0[:, None, None], axis=1)[:, 0, :]
    iota = _perm(jnp.broadcast_to(
        jnp.arange(_N, dtype=jnp.float32)[None, :], (_B, _N)))
    tri = (jnp.arange(_W, dtype=jnp.int32)[:, None]
           <= jnp.arange(_W, dtype=jnp.int32)[None, :]).astype(jnp.float32)
    out_shapes = (
        jax.ShapeDtypeStruct((_NPT, _B), jnp.float32),
        jax.ShapeDtypeStruct((_NPT, _B), jnp.float32),
        jax.ShapeDtypeStruct((_NPT, _B), jnp.float32),
        jax.ShapeDtypeStruct((_NPT, _B), jnp.float32),
    )
    idx_t, cx_t, cy_t, cz_t = pl.pallas_call(
        _fps_body,
        out_shape=out_shapes,
        scratch_shapes=[pltpu.VMEM((_B, _N), jnp.float32)],
    )(x, y, z, iota, tri, far0[:, None].astype(jnp.float32), c0)
    idx = idx_t.T.astype(jnp.int32)
    new_xyz = jnp.stack([cx_t.T, cy_t.T, cz_t.T], axis=-1)
    return (new_xyz, idx)
